# baseline (speedup 1.0000x reference)
import jax
import jax.numpy as jnp
from jax.experimental import pallas as pl
from jax.experimental.pallas import tpu as pltpu

IMG_BLOCK = 16
CHUNK_IMGS = 4
DEC_TK = 3584


def _conv_stack_kernel(x_ref, w1_ref, s1_ref, c1_ref,
                       w2_ref, s2_ref, c2_ref,
                       w3_ref, s3_ref, c3_ref,
                       o_ref, p1_ref, p2_ref, p3_ref):
    B, H, W = o_ref.shape[0], o_ref.shape[1], o_ref.shape[2]
    Hp, Wp = H + 2, W + 2

    CB = CHUNK_IMGS

    def zero_borders(p_ref):
        c = p_ref.shape[-1]
        zrow = jnp.zeros((B, 1, Wp, c), jnp.bfloat16)
        zcol = jnp.zeros((B, H, 1, c), jnp.bfloat16)
        p_ref[0:B, 0:1, :, :] = zrow
        p_ref[0:B, H + 1:H + 2, :, :] = zrow
        p_ref[0:B, 1:H + 1, 0:1, :] = zcol
        p_ref[0:B, 1:H + 1, W + 1:W + 2, :] = zcol

    def conv_layer(p_ref, w_ref, s_ref, c_ref, store):
        cin = p_ref.shape[-1]
        cout = w_ref.shape[3]
        Mc = CB * Hp * Wp
        scale = s_ref[...].reshape(1, 1, 1, cout)
        bias = c_ref[...].reshape(1, 1, 1, cout)
        wcats = [w_ref[:, dx].reshape(3 * cin, cout).astype(jnp.bfloat16)
                 for dx in range(3)]

        def body(i, _):
            i0 = i * CB
            xm = p_ref[pl.ds(i0, CB + 1)].reshape((CB + 1) * Hp * Wp, cin)
            a3w = jnp.concatenate(
                [xm[0:Mc], xm[Wp:Wp + Mc], xm[2 * Wp:2 * Wp + Mc]], axis=1)
            acc = jnp.zeros((CB, H, W, cout), jnp.float32)
            for dx in range(3):
                part = jnp.dot(a3w, wcats[dx],
                               preferred_element_type=jnp.float32)
                part = part.reshape(CB, Hp, Wp, cout)
                acc = acc + part[:, 0:H, dx:dx + W, :]
            y = jnp.maximum(acc * scale + bias, 0.0).astype(jnp.bfloat16)
            store(i0, y)
            return 0

        jax.lax.fori_loop(0, B // CB, body, 0)

    def store_pad(p_ref):
        def store(i0, y):
            p_ref[pl.ds(i0, CB), 1:H + 1, 1:W + 1, :] = y
        return store

    zero_borders(p2_ref)
    zero_borders(p3_ref)
    zero_borders(p1_ref)
    p1_ref[0:B, 1:H + 1, 1:W + 1, :] = x_ref[...].astype(jnp.bfloat16)

    conv_layer(p1_ref, w1_ref, s1_ref, c1_ref, store_pad(p2_ref))
    conv_layer(p2_ref, w2_ref, s2_ref, c2_ref, store_pad(p3_ref))

    def store_out(i0, y):
        o_ref[pl.ds(i0, CB)] = y
    conv_layer(p3_ref, w3_ref, s3_ref, c3_ref, store_out)


def _conv_stack(x_nhwc, w1, s1, c1, w2, s2, c2, w3, s3, c3):
    N, H, W, Cin = x_nhwc.shape
    Hp, Wp = H + 2, W + 2
    B = IMG_BLOCK
    return pl.pallas_call(
        _conv_stack_kernel,
        out_shape=jax.ShapeDtypeStruct((N, H, W, 128), jnp.bfloat16),
        grid=(N // B,),
        in_specs=[
            pl.BlockSpec((B, H, W, Cin), lambda n: (n, 0, 0, 0)),
            pl.BlockSpec((3, 3, Cin, 32), lambda n: (0, 0, 0, 0)),
            pl.BlockSpec((1, 32), lambda n: (0, 0)),
            pl.BlockSpec((1, 32), lambda n: (0, 0)),
            pl.BlockSpec((3, 3, 32, 64), lambda n: (0, 0, 0, 0)),
            pl.BlockSpec((1, 64), lambda n: (0, 0)),
            pl.BlockSpec((1, 64), lambda n: (0, 0)),
            pl.BlockSpec((3, 3, 64, 128), lambda n: (0, 0, 0, 0)),
            pl.BlockSpec((1, 128), lambda n: (0, 0)),
            pl.BlockSpec((1, 128), lambda n: (0, 0)),
        ],
        out_specs=pl.BlockSpec((B, H, W, 128), lambda n: (n, 0, 0, 0)),
        scratch_shapes=[
            pltpu.VMEM((B + 1, Hp, Wp, Cin), jnp.bfloat16),
            pltpu.VMEM((B + 1, Hp, Wp, 32), jnp.bfloat16),
            pltpu.VMEM((B + 1, Hp, Wp, 64), jnp.bfloat16),
        ],
        compiler_params=pltpu.CompilerParams(
            dimension_semantics=("parallel",)),
    )(x_nhwc, w1, s1, c1, w2, s2, c2, w3, s3, c3)


def _decoder_kernel(x_ref, w1_ref, b1_ref, w2_ref, o_ref, acc_ref):
    k = pl.program_id(1)

    @pl.when(k == 0)
    def _():
        acc_ref[...] = jnp.zeros_like(acc_ref)

    acc_ref[...] += jnp.dot(x_ref[...], w1_ref[0],
                            preferred_element_type=jnp.float32)

    @pl.when(k == pl.num_programs(1) - 1)
    def _():
        h = jax.nn.sigmoid(acc_ref[...] + b1_ref[...])
        o_ref[0] = jnp.dot(h, w2_ref[...],
                           preferred_element_type=jnp.float32)


def _decoder(x, dw1, db1, dw2, db2):
    B, K = x.shape
    n_half, Kw, hh = dw1.shape
    C = dw2.shape[1]
    tk = DEC_TK
    partial = pl.pallas_call(
        _decoder_kernel,
        out_shape=jax.ShapeDtypeStruct((n_half, B, C), jnp.float32),
        grid=(n_half, K // tk),
        in_specs=[
            pl.BlockSpec((B, tk), lambda h, k: (0, k)),
            pl.BlockSpec((1, tk, hh), lambda h, k: (h, k, 0)),
            pl.BlockSpec((1, hh), lambda h, k: (0, h)),
            pl.BlockSpec((hh, C), lambda h, k: (h, 0)),
        ],
        out_specs=pl.BlockSpec((1, B, C), lambda h, k: (h, 0, 0)),
        scratch_shapes=[pltpu.VMEM((B, hh), jnp.float32)],
        compiler_params=pltpu.CompilerParams(
            dimension_semantics=("parallel", "arbitrary"),
            vmem_limit_bytes=48 << 20),
    )(x, dw1, db1, dw2)
    return jnp.sum(partial, axis=0) + db2


@jax.jit
def kernel(x_nchw, w1, s1, c1, w2, s2, c2, w3, s3, c3, dw1, db1, dw2, db2):
    x = jnp.transpose(x_nchw, (0, 2, 3, 1))
    x = _conv_stack(x, w1, s1, c1, w2, s2, c2, w3, s3, c3)
    x = x.reshape(x.shape[0], -1)
    return _decoder(x, dw1, db1, dw2, db2)


# speedup vs baseline: 1.1943x; 1.1943x over previous
import jax
import jax.numpy as jnp
from jax.experimental import pallas as pl
from jax.experimental.pallas import tpu as pltpu

IMG_BLOCK = 8
CHUNK_IMGS = 8
DEC_TK = 3584


def _conv_stack_kernel(x_ref, w1_ref, s1_ref, c1_ref,
                       w2_ref, s2_ref, c2_ref,
                       w3_ref, s3_ref, c3_ref,
                       o_ref, p1_ref, p2_ref, p3_ref):
    B, H, W = o_ref.shape[0], o_ref.shape[1], o_ref.shape[2]
    Hp, Wp = H + 2, W + 2

    CB = CHUNK_IMGS

    def zero_borders(p_ref):
        c = p_ref.shape[-1]
        zrow = jnp.zeros((B, 1, Wp, c), jnp.bfloat16)
        zcol = jnp.zeros((B, H, 1, c), jnp.bfloat16)
        p_ref[0:B, 0:1, :, :] = zrow
        p_ref[0:B, H + 1:H + 2, :, :] = zrow
        p_ref[0:B, 1:H + 1, 0:1, :] = zcol
        p_ref[0:B, 1:H + 1, W + 1:W + 2, :] = zcol

    def conv_layer(p_ref, w_ref, s_ref, c_ref, store):
        cin = p_ref.shape[-1]
        cout = w_ref.shape[3]
        Mc = CB * Hp * Wp
        scale = s_ref[...].reshape(1, 1, 1, cout)
        bias = c_ref[...].reshape(1, 1, 1, cout)
        wcats = [w_ref[:, dx].reshape(3 * cin, cout).astype(jnp.bfloat16)
                 for dx in range(3)]

        def body(i, _):
            i0 = i * CB
            xm = p_ref[pl.ds(i0, CB + 1)].reshape((CB + 1) * Hp * Wp, cin)
            a3w = jnp.concatenate(
                [xm[0:Mc], xm[Wp:Wp + Mc], xm[2 * Wp:2 * Wp + Mc]], axis=1)
            acc = jnp.zeros((CB, H, W, cout), jnp.float32)
            for dx in range(3):
                part = jnp.dot(a3w, wcats[dx],
                               preferred_element_type=jnp.float32)
                part = part.reshape(CB, Hp, Wp, cout)
                acc = acc + part[:, 0:H, dx:dx + W, :]
            y = jnp.maximum(acc * scale + bias, 0.0).astype(jnp.bfloat16)
            store(i0, y)
            return 0

        jax.lax.fori_loop(0, B // CB, body, 0)

    def store_pad(p_ref):
        def store(i0, y):
            p_ref[pl.ds(i0, CB), 1:H + 1, 1:W + 1, :] = y
        return store

    zero_borders(p2_ref)
    zero_borders(p3_ref)
    zero_borders(p1_ref)
    p1_ref[0:B, 1:H + 1, 1:W + 1, :] = x_ref[...].astype(jnp.bfloat16)

    conv_layer(p1_ref, w1_ref, s1_ref, c1_ref, store_pad(p2_ref))
    conv_layer(p2_ref, w2_ref, s2_ref, c2_ref, store_pad(p3_ref))

    def store_out(i0, y):
        o_ref[pl.ds(i0, CB)] = y
    conv_layer(p3_ref, w3_ref, s3_ref, c3_ref, store_out)


def _conv_stack(x_nhwc, w1, s1, c1, w2, s2, c2, w3, s3, c3):
    N, H, W, Cin = x_nhwc.shape
    Hp, Wp = H + 2, W + 2
    B = IMG_BLOCK
    return pl.pallas_call(
        _conv_stack_kernel,
        out_shape=jax.ShapeDtypeStruct((N, H, W, 128), jnp.bfloat16),
        grid=(N // B,),
        in_specs=[
            pl.BlockSpec((B, H, W, Cin), lambda n: (n, 0, 0, 0)),
            pl.BlockSpec((3, 3, Cin, 32), lambda n: (0, 0, 0, 0)),
            pl.BlockSpec((1, 32), lambda n: (0, 0)),
            pl.BlockSpec((1, 32), lambda n: (0, 0)),
            pl.BlockSpec((3, 3, 32, 64), lambda n: (0, 0, 0, 0)),
            pl.BlockSpec((1, 64), lambda n: (0, 0)),
            pl.BlockSpec((1, 64), lambda n: (0, 0)),
            pl.BlockSpec((3, 3, 64, 128), lambda n: (0, 0, 0, 0)),
            pl.BlockSpec((1, 128), lambda n: (0, 0)),
            pl.BlockSpec((1, 128), lambda n: (0, 0)),
        ],
        out_specs=pl.BlockSpec((B, H, W, 128), lambda n: (n, 0, 0, 0)),
        scratch_shapes=[
            pltpu.VMEM((B + 1, Hp, Wp, Cin), jnp.bfloat16),
            pltpu.VMEM((B + 1, Hp, Wp, 32), jnp.bfloat16),
            pltpu.VMEM((B + 1, Hp, Wp, 64), jnp.bfloat16),
        ],
        compiler_params=pltpu.CompilerParams(
            dimension_semantics=("parallel",)),
    )(x_nhwc, w1, s1, c1, w2, s2, c2, w3, s3, c3)


def _decoder_kernel(x_ref, w1_ref, b1_ref, w2_ref, o_ref, acc_ref):
    k = pl.program_id(1)

    @pl.when(k == 0)
    def _():
        acc_ref[...] = jnp.zeros_like(acc_ref)

    acc_ref[...] += jnp.dot(x_ref[...], w1_ref[0],
                            preferred_element_type=jnp.float32)

    @pl.when(k == pl.num_programs(1) - 1)
    def _():
        h = jax.nn.sigmoid(acc_ref[...] + b1_ref[...])
        o_ref[0] = jnp.dot(h, w2_ref[...],
                           preferred_element_type=jnp.float32)


def _decoder(x, dw1, db1, dw2, db2):
    B, K = x.shape
    n_half, Kw, hh = dw1.shape
    C = dw2.shape[1]
    tk = DEC_TK
    partial = pl.pallas_call(
        _decoder_kernel,
        out_shape=jax.ShapeDtypeStruct((n_half, B, C), jnp.float32),
        grid=(n_half, K // tk),
        in_specs=[
            pl.BlockSpec((B, tk), lambda h, k: (0, k)),
            pl.BlockSpec((1, tk, hh), lambda h, k: (h, k, 0)),
            pl.BlockSpec((1, hh), lambda h, k: (0, h)),
            pl.BlockSpec((hh, C), lambda h, k: (h, 0)),
        ],
        out_specs=pl.BlockSpec((1, B, C), lambda h, k: (h, 0, 0)),
        scratch_shapes=[pltpu.VMEM((B, hh), jnp.float32)],
        compiler_params=pltpu.CompilerParams(
            dimension_semantics=("parallel", "arbitrary"),
            vmem_limit_bytes=48 << 20),
    )(x, dw1, db1, dw2)
    return jnp.sum(partial, axis=0) + db2


@jax.jit
def kernel(x_nchw, w1, s1, c1, w2, s2, c2, w3, s3, c3, dw1, db1, dw2, db2):
    x = jnp.transpose(x_nchw, (0, 2, 3, 1))
    x = _conv_stack(x, w1, s1, c1, w2, s2, c2, w3, s3, c3)
    x = x.reshape(x.shape[0], -1)
    return _decoder(x, dw1, db1, dw2, db2)
